# baseline (speedup 1.0000x reference)
import functools

import jax
import jax.numpy as jnp
from jax import lax
from jax.experimental import pallas as pl
from jax.experimental.pallas import tpu as pltpu
from jax.experimental.pallas import tpu_sc as plsc

B = 16384
L = 200
DIM = 64
BB = 128
NW = 32
BLK_PER_W = B // BB // NW
N_PAIRS = L // 2


def _make_kernel():
    mesh = plsc.VectorSubcoreMesh(core_axis_name="c", subcore_axis_name="s")

    @functools.partial(
        pl.kernel,
        out_type=jax.ShapeDtypeStruct((L, 8, BB, 8, BB), jnp.float32),
        mesh=mesh,
        scratch_types=[
            pltpu.VMEM((BB, L), jnp.int32),
            pltpu.VMEM((L * BB,), jnp.int32),
            pltpu.VMEM((BB, DIM), jnp.float32),
            pltpu.VMEM((BB, DIM), jnp.float32),
            pltpu.VMEM((1, 8, 1, 8, BB), jnp.float32),
            pltpu.VMEM((1, 8, 1, 8, BB), jnp.float32),
            pltpu.SemaphoreType.DMA,
            pltpu.SemaphoreType.DMA,
            pltpu.SemaphoreType.DMA,
            pltpu.SemaphoreType.DMA,
        ],
        compiler_params=pltpu.CompilerParams(
            use_tc_tiling_on_sc=False,
            skip_device_barrier=True,
            needs_layout_passes=False,
        ),
    )
    def gather_kernel(idx_hbm, table_hbm, out_hbm, idxblk, idx_t, r0, r1,
                      t0, t1, gsem0, gsem1, ssem0, ssem1):
        wid = lax.axis_index("s") * 2 + lax.axis_index("c")

        rbuf = (r0, r1)
        tbuf = (t0, t1)
        gsem = (gsem0, gsem1)
        ssem = (ssem0, ssem1)
        lanes = lax.iota(jnp.int32, 16)

        def transpose_indices():
            def l_body(l, carry):
                for k in range(BB // 16):
                    rows = k * 16 + lanes
                    v = plsc.load_gather(idxblk, [rows, jnp.full((16,), l,
                                                                 jnp.int32)])
                    idx_t[pl.ds(l * BB + k * 16, 16)] = v
                return carry
            lax.fori_loop(0, L, l_body, 0)

        def start_gather(l, b):
            pltpu.async_copy(
                table_hbm.at[idx_t.at[pl.ds(l * BB, BB)]],
                rbuf[b],
                gsem[b],
            )

        def wait_gather(b):
            pltpu.make_async_copy(
                table_hbm.at[pl.ds(0, BB)], rbuf[b], gsem[b]
            ).wait()

        def transpose_block(b):
            src = rbuf[b]
            dst = tbuf[b]

            def d_body(d, carry):
                tr = d // 8
                r = d % 8
                for k in range(BB // 16):
                    cols = k * 16 + lanes
                    v = plsc.load_gather(
                        src, [cols, jnp.full((16,), d, jnp.int32)])
                    dst[0, tr, 0, r, pl.ds(k * 16, 16)] = v
                return carry
            lax.fori_loop(0, DIM, d_body, 0)

        def start_store(l, tc, b):
            pltpu.async_copy(
                tbuf[b],
                out_hbm.at[pl.ds(l, 1), :, pl.ds(tc, 1), :, :],
                ssem[b],
            )

        def wait_store(b):
            pltpu.make_async_copy(
                out_hbm.at[pl.ds(0, 1), :, pl.ds(0, 1), :, :], tbuf[b],
                ssem[b],
            ).wait()

        for blk in range(BLK_PER_W):
            tc = wid * BLK_PER_W + blk
            pltpu.sync_copy(idx_hbm.at[pl.ds(tc * BB, BB), :], idxblk)
            transpose_indices()
            start_gather(0, 0)

            def step(l, b, nb, first_pair, last_pair):
                if not last_pair:
                    start_gather(l + 1, nb)
                wait_gather(b)
                if not first_pair:
                    wait_store(b)
                transpose_block(b)
                start_store(l, tc, b)

            def pair_body(p, carry):
                l = 2 * p

                @pl.when(p == 0)
                def _():
                    step(l, 0, 1, True, False)
                    step(l + 1, 1, 0, True, False)

                @pl.when(jnp.logical_and(p > 0, p < N_PAIRS - 1))
                def _():
                    step(l, 0, 1, False, False)
                    step(l + 1, 1, 0, False, False)

                @pl.when(p == N_PAIRS - 1)
                def _():
                    step(l, 0, 1, False, False)
                    step(l + 1, 1, 0, False, True)

                return carry

            lax.fori_loop(0, N_PAIRS, pair_body, 0)
            wait_store(0)
            wait_store(1)

    return gather_kernel


_gather = _make_kernel()


@jax.jit
def kernel(x, weight):
    idx = x.astype(jnp.int32)
    out5 = _gather(idx, weight)
    return out5.transpose(2, 4, 0, 1, 3).reshape(B, L, DIM)


# speedup vs baseline: 2.4217x; 2.4217x over previous
import functools

import jax
import jax.numpy as jnp
from jax import lax
from jax.experimental import pallas as pl
from jax.experimental.pallas import tpu as pltpu
from jax.experimental.pallas import tpu_sc as plsc

B = 16384
L = 200
DIM = 64
BB = 128
NW = 32
BLK_PER_W = B // BB // NW
N_PAIRS = L // 2


def _make_kernel():
    mesh = plsc.VectorSubcoreMesh(core_axis_name="c", subcore_axis_name="s")

    @functools.partial(
        pl.kernel,
        out_type=jax.ShapeDtypeStruct((L, 8, BB, 8, BB), jnp.float32),
        mesh=mesh,
        scratch_types=[
            pltpu.VMEM((BB, L), jnp.int32),
            pltpu.VMEM((L * BB,), jnp.int32),
            pltpu.VMEM((BB, DIM), jnp.float32),
            pltpu.VMEM((BB, DIM), jnp.float32),
            pltpu.VMEM((1, 8, 1, 8, BB + 1), jnp.float32),
            pltpu.VMEM((1, 8, 1, 8, BB + 1), jnp.float32),
            pltpu.SemaphoreType.DMA,
            pltpu.SemaphoreType.DMA,
            pltpu.SemaphoreType.DMA,
            pltpu.SemaphoreType.DMA,
        ],
        compiler_params=pltpu.CompilerParams(
            use_tc_tiling_on_sc=False,
            skip_device_barrier=True,
            needs_layout_passes=False,
        ),
    )
    def gather_kernel(idx_hbm, table_hbm, out_hbm, idxblk, idx_t, r0, r1,
                      t0, t1, gsem0, gsem1, ssem0, ssem1):
        wid = lax.axis_index("s") * 2 + lax.axis_index("c")

        rbuf = (r0, r1)
        tbuf = (t0, t1)
        gsem = (gsem0, gsem1)
        ssem = (ssem0, ssem1)
        lanes = lax.iota(jnp.int32, 16)

        def transpose_indices():
            def l_body(l, carry):
                for k in range(BB // 16):
                    rows = k * 16 + lanes
                    v = plsc.load_gather(idxblk, [rows, jnp.full((16,), l,
                                                                 jnp.int32)])
                    idx_t[pl.ds(l * BB + k * 16, 16)] = v
                return carry
            lax.fori_loop(0, L, l_body, 0)

        def start_gather(l, b):
            pltpu.async_copy(
                table_hbm.at[idx_t.at[pl.ds(l * BB, BB)]],
                rbuf[b],
                gsem[b],
            )

        def wait_gather(b):
            pltpu.make_async_copy(
                table_hbm.at[pl.ds(0, BB)], rbuf[b], gsem[b]
            ).wait()

        zeros16 = jnp.zeros((16,), jnp.int32)
        trk = [2 * k + lanes // 8 for k in range(DIM // 16)]
        rk = [lanes % 8 for _ in range(DIM // 16)]

        def transpose_block(b):
            src = rbuf[b]
            dst = tbuf[b]

            def c_body(c, carry):
                cvec = jnp.full((16,), c, jnp.int32)
                for k in range(DIM // 16):
                    v = src[c, pl.ds(16 * k, 16)]
                    plsc.store_scatter(
                        dst, [zeros16, trk[k], zeros16, rk[k], cvec], v)
                return carry
            lax.fori_loop(0, BB, c_body, 0)

        def start_store(l, tc, b):
            pltpu.async_copy(
                tbuf[b].at[:, :, :, :, pl.ds(0, BB)],
                out_hbm.at[pl.ds(l, 1), :, pl.ds(tc, 1), :, :],
                ssem[b],
            )

        def wait_store(b):
            pltpu.make_async_copy(
                out_hbm.at[pl.ds(0, 1), :, pl.ds(0, 1), :, :],
                tbuf[b].at[:, :, :, :, pl.ds(0, BB)],
                ssem[b],
            ).wait()

        for blk in range(BLK_PER_W):
            tc = wid * BLK_PER_W + blk
            pltpu.sync_copy(idx_hbm.at[pl.ds(tc * BB, BB), :], idxblk)
            transpose_indices()
            start_gather(0, 0)

            def step(l, b, nb, first_pair, last_pair):
                if not last_pair:
                    start_gather(l + 1, nb)
                wait_gather(b)
                if not first_pair:
                    wait_store(b)
                transpose_block(b)
                start_store(l, tc, b)

            def pair_body(p, carry):
                l = 2 * p

                @pl.when(p == 0)
                def _():
                    step(l, 0, 1, True, False)
                    step(l + 1, 1, 0, True, False)

                @pl.when(jnp.logical_and(p > 0, p < N_PAIRS - 1))
                def _():
                    step(l, 0, 1, False, False)
                    step(l + 1, 1, 0, False, False)

                @pl.when(p == N_PAIRS - 1)
                def _():
                    step(l, 0, 1, False, False)
                    step(l + 1, 1, 0, False, True)

                return carry

            lax.fori_loop(0, N_PAIRS, pair_body, 0)
            wait_store(0)
            wait_store(1)

    return gather_kernel


_gather = _make_kernel()


@jax.jit
def kernel(x, weight):
    idx = x.astype(jnp.int32)
    out5 = _gather(idx, weight)
    return out5.transpose(2, 4, 0, 1, 3).reshape(B, L, DIM)


# speedup vs baseline: 2.4884x; 1.0275x over previous
import functools

import jax
import jax.numpy as jnp
from jax import lax
from jax.experimental import pallas as pl
from jax.experimental.pallas import tpu as pltpu
from jax.experimental.pallas import tpu_sc as plsc

B = 16384
L = 200
DIM = 64
BB = 128
NW = 32
BLK_PER_W = B // BB // NW
N_PAIRS = L // 2


def _make_kernel():
    mesh = plsc.VectorSubcoreMesh(core_axis_name="c", subcore_axis_name="s")

    @functools.partial(
        pl.kernel,
        out_type=jax.ShapeDtypeStruct((L, 8, BB, 8, BB), jnp.float32),
        mesh=mesh,
        scratch_types=[
            pltpu.VMEM((BB, L), jnp.int32),
            pltpu.VMEM((L * BB,), jnp.int32),
            pltpu.VMEM((BB, DIM), jnp.float32),
            pltpu.VMEM((BB, DIM), jnp.float32),
            pltpu.VMEM((1, 8, 1, 8, BB + 8), jnp.float32),
            pltpu.VMEM((1, 8, 1, 8, BB + 8), jnp.float32),
            pltpu.SemaphoreType.DMA,
            pltpu.SemaphoreType.DMA,
            pltpu.SemaphoreType.DMA,
            pltpu.SemaphoreType.DMA,
        ],
        compiler_params=pltpu.CompilerParams(
            use_tc_tiling_on_sc=False,
            skip_device_barrier=True,
            needs_layout_passes=False,
        ),
    )
    def gather_kernel(idx_hbm, table_hbm, out_hbm, idxblk, idx_t, r0, r1,
                      t0, t1, gsem0, gsem1, ssem0, ssem1):
        wid = lax.axis_index("s") * 2 + lax.axis_index("c")

        rbuf = (r0, r1)
        tbuf = (t0, t1)
        gsem = (gsem0, gsem1)
        ssem = (ssem0, ssem1)
        lanes = lax.iota(jnp.int32, 16)

        def transpose_indices():
            def l_body(l, carry):
                for k in range(BB // 16):
                    rows = k * 16 + lanes
                    v = plsc.load_gather(idxblk, [rows, jnp.full((16,), l,
                                                                 jnp.int32)])
                    idx_t[pl.ds(l * BB + k * 16, 16)] = v
                return carry
            lax.fori_loop(0, L, l_body, 0)

        def start_gather(l, b):
            pltpu.async_copy(
                table_hbm.at[idx_t.at[pl.ds(l * BB, BB)]],
                rbuf[b],
                gsem[b],
            )

        def wait_gather(b):
            pltpu.make_async_copy(
                table_hbm.at[pl.ds(0, BB)], rbuf[b], gsem[b]
            ).wait()

        zeros16 = jnp.zeros((16,), jnp.int32)
        trk = [2 * k + lanes // 8 for k in range(DIM // 16)]
        rk = [lanes % 8 for _ in range(DIM // 16)]

        def transpose_block(b):
            src = rbuf[b]
            dst = tbuf[b]

            def c_body(cc, carry):
                for u in range(2):
                    c = 2 * cc + u
                    cvec = jnp.full((16,), c, jnp.int32)
                    for k in range(DIM // 16):
                        v = src[c, pl.ds(16 * k, 16)]
                        plsc.store_scatter(
                            dst, [zeros16, trk[k], zeros16, rk[k], cvec], v)
                return carry
            lax.fori_loop(0, BB // 2, c_body, 0)

        def start_store(l, tc, b):
            pltpu.async_copy(
                tbuf[b].at[:, :, :, :, pl.ds(0, BB)],
                out_hbm.at[pl.ds(l, 1), :, pl.ds(tc, 1), :, :],
                ssem[b],
            )

        def wait_store(b):
            pltpu.make_async_copy(
                out_hbm.at[pl.ds(0, 1), :, pl.ds(0, 1), :, :],
                tbuf[b].at[:, :, :, :, pl.ds(0, BB)],
                ssem[b],
            ).wait()

        for blk in range(BLK_PER_W):
            tc = wid * BLK_PER_W + blk
            pltpu.sync_copy(idx_hbm.at[pl.ds(tc * BB, BB), :], idxblk)
            transpose_indices()
            start_gather(0, 0)

            def step(l, b, nb, first_pair, last_pair):
                if not last_pair:
                    start_gather(l + 1, nb)
                wait_gather(b)
                if not first_pair:
                    wait_store(b)
                transpose_block(b)
                start_store(l, tc, b)

            def pair_body(p, carry):
                l = 2 * p

                @pl.when(p == 0)
                def _():
                    step(l, 0, 1, True, False)
                    step(l + 1, 1, 0, True, False)

                @pl.when(jnp.logical_and(p > 0, p < N_PAIRS - 1))
                def _():
                    step(l, 0, 1, False, False)
                    step(l + 1, 1, 0, False, False)

                @pl.when(p == N_PAIRS - 1)
                def _():
                    step(l, 0, 1, False, False)
                    step(l + 1, 1, 0, False, True)

                return carry

            lax.fori_loop(0, N_PAIRS, pair_body, 0)
            wait_store(0)
            wait_store(1)

    return gather_kernel


_gather = _make_kernel()


@jax.jit
def kernel(x, weight):
    idx = x.astype(jnp.int32)
    out5 = _gather(idx, weight)
    return out5.transpose(2, 4, 0, 1, 3).reshape(B, L, DIM)


# speedup vs baseline: 3.1353x; 1.2600x over previous
import functools

import jax
import jax.numpy as jnp
from jax import lax
from jax.experimental import pallas as pl
from jax.experimental.pallas import tpu as pltpu
from jax.experimental.pallas import tpu_sc as plsc

B = 16384
L = 200
DIM = 64
BB = 128
NW = 32
BLK_PER_W = B // BB // NW
N_PAIRS = L // 2


def _make_kernel():
    mesh = plsc.VectorSubcoreMesh(core_axis_name="c", subcore_axis_name="s")

    @functools.partial(
        pl.kernel,
        out_type=jax.ShapeDtypeStruct((L, 8, BB, 8, BB), jnp.float32),
        mesh=mesh,
        scratch_types=[
            pltpu.VMEM((BB, L), jnp.int32),
            pltpu.VMEM((L * BB,), jnp.int32),
            pltpu.VMEM((BB, DIM), jnp.float32),
            pltpu.VMEM((BB, DIM), jnp.float32),
            pltpu.VMEM((1, 8, 1, 8, BB + 8), jnp.float32),
            pltpu.VMEM((1, 8, 1, 8, BB + 8), jnp.float32),
            pltpu.SemaphoreType.DMA,
            pltpu.SemaphoreType.DMA,
            pltpu.SemaphoreType.DMA,
            pltpu.SemaphoreType.DMA,
        ],
        compiler_params=pltpu.CompilerParams(
            use_tc_tiling_on_sc=False,
            skip_device_barrier=True,
            needs_layout_passes=False,
        ),
    )
    def gather_kernel(idx_hbm, table_hbm, out_hbm, idxblk, idx_t, r0, r1,
                      t0, t1, gsem0, gsem1, ssem0, ssem1):
        wid = lax.axis_index("s") * 2 + lax.axis_index("c")

        rbuf = (r0, r1)
        tbuf = (t0, t1)
        gsem = (gsem0, gsem1)
        ssem = (ssem0, ssem1)
        lanes = lax.iota(jnp.int32, 16)

        def transpose_indices():
            def l_body(l, carry):
                for k in range(BB // 16):
                    rows = k * 16 + lanes
                    v = plsc.load_gather(idxblk, [rows, jnp.full((16,), l,
                                                                 jnp.int32)])
                    idx_t[pl.ds(l * BB + k * 16, 16)] = v
                return carry
            lax.fori_loop(0, L, l_body, 0)

        def start_gather(l, b):
            pltpu.async_copy(
                table_hbm.at[idx_t.at[pl.ds(l * BB, BB)]],
                rbuf[b],
                gsem[b],
            )

        def wait_gather(b):
            pltpu.make_async_copy(
                table_hbm.at[pl.ds(0, BB)], rbuf[b], gsem[b]
            ).wait()

        zeros16 = jnp.zeros((16,), jnp.int32)
        trk = [2 * k + lanes // 8 for k in range(DIM // 16)]
        rk = [lanes % 8 for _ in range(DIM // 16)]

        def transpose_block(b):
            src = rbuf[b]
            dst = tbuf[b]

            def c_body(cc, carry):
                for u in range(2):
                    c = 2 * cc + u
                    cvec = jnp.full((16,), c, jnp.int32)
                    vs = [src[c, pl.ds(16 * k, 16)]
                          for k in range(DIM // 16)]
                    for k in range(DIM // 16):
                        plsc.store_scatter(
                            dst, [zeros16, trk[k], zeros16, rk[k], cvec],
                            vs[k])
                return carry
            lax.fori_loop(0, BB // 2, c_body, 0)

        def start_store(l, tc, b):
            pltpu.async_copy(
                tbuf[b].at[:, :, :, :, pl.ds(0, BB)],
                out_hbm.at[pl.ds(l, 1), :, pl.ds(tc, 1), :, :],
                ssem[b],
            )

        def wait_store(b):
            pltpu.make_async_copy(
                out_hbm.at[pl.ds(0, 1), :, pl.ds(0, 1), :, :],
                tbuf[b].at[:, :, :, :, pl.ds(0, BB)],
                ssem[b],
            ).wait()

        for blk in range(BLK_PER_W):
            tc = wid * BLK_PER_W + blk
            pltpu.sync_copy(idx_hbm.at[pl.ds(tc * BB, BB), :], idxblk)
            transpose_indices()
            start_gather(0, 0)

            def step(l, b, nb, first_pair, last_pair):
                if not last_pair:
                    start_gather(l + 1, nb)
                wait_gather(b)
                if not first_pair:
                    wait_store(b)
                transpose_block(b)
                start_store(l, tc, b)

            def pair_body(p, carry):
                l = 2 * p

                @pl.when(p == 0)
                def _():
                    step(l, 0, 1, True, False)
                    step(l + 1, 1, 0, True, False)

                @pl.when(jnp.logical_and(p > 0, p < N_PAIRS - 1))
                def _():
                    step(l, 0, 1, False, False)
                    step(l + 1, 1, 0, False, False)

                @pl.when(p == N_PAIRS - 1)
                def _():
                    step(l, 0, 1, False, False)
                    step(l + 1, 1, 0, False, True)

                return carry

            lax.fori_loop(0, N_PAIRS, pair_body, 0)
            wait_store(0)
            wait_store(1)

    return gather_kernel


_gather = _make_kernel()


@jax.jit
def kernel(x, weight):
    idx = x.astype(jnp.int32)
    out5 = _gather(idx, weight)
    return out5.transpose(2, 4, 0, 1, 3).reshape(B, L, DIM)
